# baseline (speedup 1.0000x reference)
import functools
import jax
import jax.numpy as jnp
from jax import lax
from jax.experimental import pallas as pl
from jax.experimental.pallas import tpu as pltpu
from jax.experimental.pallas import tpu_sc as plsc

N = 10000
E = 320000
F = 128
C = 40
CP = 48

NC = 2
NS = 16
NW = NC * NS
K = 128
CH = 80
CPP = 2 * CH
TOTC = NW * CH
TOTCP = 2624
EPAD = TOTCP * K
NP = 10240
RPS = NP // NS
CHD = TOTCP // NW

RB = 400
G = N // RB

f32 = jnp.float32
i32 = jnp.int32


def _deg_body(dst_hbm, ew_hbm, out_hbm, dst_v, ew_v, acc_v):
    c = lax.axis_index("c")
    s = lax.axis_index("s")
    wid = s * NC + c
    pltpu.sync_copy(dst_hbm.at[pl.ds(wid * CHD, CHD)], dst_v)
    pltpu.sync_copy(ew_hbm.at[pl.ds(wid * CHD, CHD)], ew_v)

    def init(i, carry):
        acc_v[pl.ds(i * 16, 16)] = jnp.zeros((16,), f32)
        return carry
    lax.fori_loop(0, NP // 16, init, 0)

    def chunk(ci, carry):
        def grp(g, carry2):
            d16 = dst_v[ci, pl.ds(g * 16, 16)]
            w16 = ew_v[ci, pl.ds(g * 16, 16)]
            plsc.addupdate_scatter(acc_v, [d16], w16)
            return carry2
        return lax.fori_loop(0, K // 16, grp, carry)
    lax.fori_loop(0, CHD, chunk, 0)

    pltpu.sync_copy(acc_v, out_hbm.at[wid, 0])


_deg_call = pl.kernel(
    _deg_body,
    out_type=jax.ShapeDtypeStruct((NW, 1, NP), f32),
    mesh=plsc.VectorSubcoreMesh(core_axis_name="c", subcore_axis_name="s"),
    compiler_params=pltpu.CompilerParams(needs_layout_passes=False,
                                         use_tc_tiling_on_sc=False),
    scratch_types=[
        pltpu.VMEM((CHD, K), i32),
        pltpu.VMEM((CHD, K), f32),
        pltpu.VMEM((NP,), f32),
    ],
)


def _agg_body(fd, H, n0, *refs):
    qs = refs[:H]
    (src_hbm, dst_hbm, ew_hbm, out_hbm, src_v, dst_v, ew_v,
     rows0, rows1, msg0, msg1, zbuf, acc_sh, gs0, gs1, ss0, ss1) = refs[H:]
    rows = (rows0, rows1)
    msg = (msg0, msg1)
    gsem = (gs0, gs1)
    ssem = (ss0, ss1)
    c = lax.axis_index("c")
    s = lax.axis_index("s")
    fb_n = fd // 16
    n1 = CPP - n0
    base = jnp.where(c == 0, s * n0, NS * n0 + s * n1)
    nch = jnp.where(c == 0, n0, n1)

    pltpu.sync_copy(src_hbm.at[pl.ds(base, n0)], src_v)
    pltpu.sync_copy(dst_hbm.at[pl.ds(base, n0)], dst_v)
    pltpu.sync_copy(ew_hbm.at[pl.ds(base, n0)], ew_v)

    def zrow(r, carry):
        for fb in range(fb_n):
            zbuf[r, pl.ds(fb * 16, 16)] = jnp.zeros((16,), f32)
        return carry
    lax.fori_loop(0, K, zrow, 0)

    for h in range(H):
        for j in range(RPS // K):
            pltpu.sync_copy(zbuf, acc_sh.at[pl.ds(s * RPS + j * K, K)])
        plsc.subcore_barrier()

        for b in range(2):
            pltpu.async_copy(qs[h].at[src_v.at[b]], rows[b], gsem[b])

        def pair(p, carry):
            for b in range(2):
                ci = 2 * p + b
                pltpu.make_async_copy(qs[h].at[src_v.at[ci]],
                                      rows[b], gsem[b]).wait()
                @pl.when(ci >= 2)
                def _drain():
                    pltpu.make_async_copy(
                        msg[b], acc_sh.at[dst_v.at[ci - 2]], ssem[b]).wait()

                @plsc.parallel_loop(0, K, unroll=8)
                def _scale(e):
                    splat = plsc.load_gather(
                        ew_v,
                        [jnp.full((16,), ci, i32), jnp.full((16,), e, i32)])
                    for fb in range(fb_n):
                        sl = pl.ds(fb * 16, 16)
                        msg[b][e, sl] = splat * rows[b][e, sl]

                @pl.when(ci + 2 < nch)
                def _prefetch():
                    pltpu.async_copy(qs[h].at[src_v.at[ci + 2]],
                                     rows[b], gsem[b])

                pltpu.async_copy(msg[b], acc_sh.at[dst_v.at[ci]], ssem[b],
                                 add=True)
            return carry
        lax.fori_loop(0, nch // 2, pair, 0)
        for b in range(2):
            pltpu.make_async_copy(msg[b], acc_sh.at[dst_v.at[nch - 2 + b]],
                                  ssem[b]).wait()

        plsc.subcore_barrier()
        pltpu.sync_copy(acc_sh.at[pl.ds(s * RPS, RPS)],
                        out_hbm.at[h, c, pl.ds(s * RPS, RPS)])
        plsc.subcore_barrier()


def _make_agg(fd, H, n0):
    return pl.kernel(
        functools.partial(_agg_body, fd, H, n0),
        out_type=jax.ShapeDtypeStruct((H, NC, NP, fd), f32),
        mesh=plsc.VectorSubcoreMesh(core_axis_name="c", subcore_axis_name="s"),
        compiler_params=pltpu.CompilerParams(needs_layout_passes=False,
                                             use_tc_tiling_on_sc=False),
        scratch_types=[
            pltpu.VMEM((n0, K), i32),
            pltpu.VMEM((n0, K), i32),
            pltpu.VMEM((n0, K), f32),
            pltpu.VMEM((K, fd), f32),
            pltpu.VMEM((K, fd), f32),
            pltpu.VMEM((K, fd), f32),
            pltpu.VMEM((K, fd), f32),
            pltpu.VMEM((K, fd), f32),
            pltpu.VMEM_SHARED((NP, fd), f32),
            pltpu.SemaphoreType.DMA,
            pltpu.SemaphoreType.DMA,
            pltpu.SemaphoreType.DMA,
            pltpu.SemaphoreType.DMA,
        ],
    )


FH = F // 2
_agg1_call = _make_agg(FH, 2, 80)
_agg2_call = _make_agg(CP, 1, 80)


def _ln(xb, g, b):
    mu = jnp.mean(xb, axis=1, keepdims=True)
    xc = xb - mu
    var = jnp.mean(xc * xc, axis=1, keepdims=True)
    return xc * lax.rsqrt(var + 1e-5) * g + b


def _tcd_body(degp_ref, o_ref):
    o_ref[...] = lax.rsqrt(1.0 + jnp.sum(degp_ref[...], axis=0,
                                         keepdims=True))


def _tc1_body(x_ref, g_ref, b_ref, w_ref, o_ref):
    h = _ln(x_ref[...], g_ref[...], b_ref[...])
    o_ref[...] = lax.dot_general(h, w_ref[...], (((1,), (1,)), ((), ())),
                                 preferred_element_type=f32)


def _tc2_body(dinv_ref, p_ref, o_ref):
    o_ref[...] = p_ref[...] * dinv_ref[...]


def _tc3_body(dinv_ref, agg_ref, q_ref, b1_ref, g_ref, bl_ref, w2_ref, o_ref):
    dinv = dinv_ref[...]
    agg = jnp.concatenate(
        [agg_ref[0, 0] + agg_ref[0, 1], agg_ref[1, 0] + agg_ref[1, 1]], axis=1)
    h = jnp.maximum(dinv * (agg + q_ref[...]) + b1_ref[...], 0.0)
    h = _ln(h, g_ref[...], bl_ref[...])
    p2 = lax.dot_general(h, w2_ref[...], (((1,), (1,)), ((), ())),
                         preferred_element_type=f32)
    o_ref[...] = p2 * dinv


def _tc4_body(dinv_ref, agg_ref, q_ref, b2_ref, o_ref):
    dinv = dinv_ref[...]
    o = dinv * (agg_ref[0, 0] + agg_ref[0, 1] + q_ref[...])
    o = o[:, :C] + b2_ref[...]
    m = jnp.max(o, axis=1, keepdims=True)
    sh = o - m
    o_ref[...] = sh - jnp.log(jnp.sum(jnp.exp(sh), axis=1, keepdims=True))


def _row_spec(fd):
    return pl.BlockSpec((RB, fd), lambda i: (i, 0))


_deg_spec = pl.BlockSpec((RB, 1), lambda i: (i, 0))
_vec_spec_f = pl.BlockSpec((1, F), lambda i: (0, 0))

DCB = 1280

_tcd_call = pl.pallas_call(
    _tcd_body, grid=(NP // DCB,),
    in_specs=[pl.BlockSpec((NW, DCB), lambda j: (0, j))],
    out_specs=pl.BlockSpec((1, DCB), lambda j: (0, j)),
    out_shape=jax.ShapeDtypeStruct((1, NP), f32),
)

_tc1_call = pl.pallas_call(
    _tc1_body, grid=(G,),
    in_specs=[_row_spec(F), _vec_spec_f, _vec_spec_f,
              pl.BlockSpec((F, F), lambda i: (0, 0))],
    out_specs=_row_spec(F),
    out_shape=jax.ShapeDtypeStruct((N, F), f32),
)

_tc2_call = pl.pallas_call(
    _tc2_body, grid=(G,),
    in_specs=[_deg_spec, _row_spec(F)],
    out_specs=_row_spec(F),
    out_shape=jax.ShapeDtypeStruct((N, F), f32),
)

_tc3_call = pl.pallas_call(
    _tc3_body, grid=(G,),
    in_specs=[_deg_spec,
              pl.BlockSpec((2, NC, RB, FH), lambda i: (0, 0, i, 0)),
              _row_spec(F), _vec_spec_f, _vec_spec_f, _vec_spec_f,
              pl.BlockSpec((CP, F), lambda i: (0, 0))],
    out_specs=_row_spec(CP),
    out_shape=jax.ShapeDtypeStruct((N, CP), f32),
)

_tc4_call = pl.pallas_call(
    _tc4_body, grid=(G,),
    in_specs=[_deg_spec,
              pl.BlockSpec((1, NC, RB, CP), lambda i: (0, 0, i, 0)),
              _row_spec(CP),
              pl.BlockSpec((1, C), lambda i: (0, 0))],
    out_specs=_row_spec(C),
    out_shape=jax.ShapeDtypeStruct((N, C), f32),
)


def kernel(x, edge_index, edge_weight, ln0_g, ln0_b, W1, b1,
           ln1_g, ln1_b, W2, b2):
    pad = EPAD - E
    src = jnp.concatenate([edge_index[0], jnp.zeros((pad,), i32)])
    dst = jnp.concatenate([edge_index[1], jnp.zeros((pad,), i32)])
    ew = jnp.concatenate([edge_weight, jnp.zeros((pad,), f32)])
    srcp = src.reshape(TOTCP, K)
    dstp = dst.reshape(TOTCP, K)
    ewp = ew.reshape(TOTCP, K)
    W2p = jnp.concatenate([W2, jnp.zeros((CP - C, F), f32)], axis=0)

    degp = _deg_call(dstp, ewp).reshape(NW, NP)
    dinv = _tcd_call(degp).reshape(NP, 1)
    p1 = _tc1_call(x, ln0_g.reshape(1, F), ln0_b.reshape(1, F), W1)
    q1 = _tc2_call(dinv, p1)
    agg1 = _agg1_call(q1[:, :FH], q1[:, FH:], srcp, dstp, ewp)
    q2 = _tc3_call(dinv, agg1, q1, b1.reshape(1, F), ln1_g.reshape(1, F),
                   ln1_b.reshape(1, F), W2p)
    agg2 = _agg2_call(q2, srcp, dstp, ewp)
    return _tc4_call(dinv, agg2, q2, b2.reshape(1, C))


# speedup vs baseline: 1.0005x; 1.0005x over previous
import functools
import jax
import jax.numpy as jnp
from jax import lax
from jax.experimental import pallas as pl
from jax.experimental.pallas import tpu as pltpu
from jax.experimental.pallas import tpu_sc as plsc

N = 10000
E = 320000
F = 128
C = 40
CP = 48

NC = 2
NS = 16
NW = NC * NS
K = 128
CH = 80
CPP = 2 * CH
TOTC = NW * CH
TOTCP = 2624
EPAD = TOTCP * K
NP = 10240
RPS = NP // NS
CHD = TOTCP // NW

RB = 400
G = N // RB

f32 = jnp.float32
i32 = jnp.int32


def _deg_body(dst_hbm, ew_hbm, out_hbm, dst_v, ew_v, acc_v):
    c = lax.axis_index("c")
    s = lax.axis_index("s")
    wid = s * NC + c
    pltpu.sync_copy(dst_hbm.at[pl.ds(wid * CHD, CHD)], dst_v)
    pltpu.sync_copy(ew_hbm.at[pl.ds(wid * CHD, CHD)], ew_v)

    def init(i, carry):
        acc_v[pl.ds(i * 16, 16)] = jnp.zeros((16,), f32)
        return carry
    lax.fori_loop(0, NP // 16, init, 0)

    def chunk(ci, carry):
        def grp(g, carry2):
            d16 = dst_v[ci, pl.ds(g * 16, 16)]
            w16 = ew_v[ci, pl.ds(g * 16, 16)]
            plsc.addupdate_scatter(acc_v, [d16], w16)
            return carry2
        return lax.fori_loop(0, K // 16, grp, carry)
    lax.fori_loop(0, CHD, chunk, 0)

    pltpu.sync_copy(acc_v, out_hbm.at[wid, 0])


_deg_call = pl.kernel(
    _deg_body,
    out_type=jax.ShapeDtypeStruct((NW, 1, NP), f32),
    mesh=plsc.VectorSubcoreMesh(core_axis_name="c", subcore_axis_name="s"),
    compiler_params=pltpu.CompilerParams(needs_layout_passes=False,
                                         use_tc_tiling_on_sc=False),
    scratch_types=[
        pltpu.VMEM((CHD, K), i32),
        pltpu.VMEM((CHD, K), f32),
        pltpu.VMEM((NP,), f32),
    ],
)


def _agg_body(fd, H, n0, *refs):
    qs = refs[:H]
    (src_hbm, dst_hbm, ew_hbm, out_hbm, src_v, dst_v, ew_v,
     rows0, rows1, msg0, msg1, zbuf, acc_sh, gs0, gs1, ss0, ss1) = refs[H:]
    rows = (rows0, rows1)
    msg = (msg0, msg1)
    gsem = (gs0, gs1)
    ssem = (ss0, ss1)
    c = lax.axis_index("c")
    s = lax.axis_index("s")
    fb_n = fd // 16
    n1 = CPP - n0
    if n0 == n1:
        base = (s * NC + c) * n0
        nch = n0
    else:
        base = jnp.where(c == 0, s * n0, NS * n0 + s * n1)
        nch = jnp.where(c == 0, n0, n1)

    pltpu.sync_copy(src_hbm.at[pl.ds(base, n0)], src_v)
    pltpu.sync_copy(dst_hbm.at[pl.ds(base, n0)], dst_v)
    pltpu.sync_copy(ew_hbm.at[pl.ds(base, n0)], ew_v)

    def zrow(r, carry):
        for fb in range(fb_n):
            zbuf[r, pl.ds(fb * 16, 16)] = jnp.zeros((16,), f32)
        return carry
    lax.fori_loop(0, K, zrow, 0)

    for h in range(H):
        for j in range(RPS // K):
            pltpu.sync_copy(zbuf, acc_sh.at[pl.ds(s * RPS + j * K, K)])
        plsc.subcore_barrier()

        for b in range(2):
            pltpu.async_copy(qs[h].at[src_v.at[b]], rows[b], gsem[b])

        def pair(p, carry):
            for b in range(2):
                ci = 2 * p + b
                pltpu.make_async_copy(qs[h].at[src_v.at[ci]],
                                      rows[b], gsem[b]).wait()
                @pl.when(ci >= 2)
                def _drain():
                    pltpu.make_async_copy(
                        msg[b], acc_sh.at[dst_v.at[ci - 2]], ssem[b]).wait()

                @plsc.parallel_loop(0, K, unroll=8)
                def _scale(e):
                    splat = plsc.load_gather(
                        ew_v,
                        [jnp.full((16,), ci, i32), jnp.full((16,), e, i32)])
                    for fb in range(fb_n):
                        sl = pl.ds(fb * 16, 16)
                        msg[b][e, sl] = splat * rows[b][e, sl]

                @pl.when(ci + 2 < nch)
                def _prefetch():
                    pltpu.async_copy(qs[h].at[src_v.at[ci + 2]],
                                     rows[b], gsem[b])

                pltpu.async_copy(msg[b], acc_sh.at[dst_v.at[ci]], ssem[b],
                                 add=True)
            return carry
        lax.fori_loop(0, nch // 2, pair, 0)
        for b in range(2):
            pltpu.make_async_copy(msg[b], acc_sh.at[dst_v.at[nch - 2 + b]],
                                  ssem[b]).wait()

        plsc.subcore_barrier()
        pltpu.sync_copy(acc_sh.at[pl.ds(s * RPS, RPS)],
                        out_hbm.at[h, c, pl.ds(s * RPS, RPS)])
        plsc.subcore_barrier()


def _make_agg(fd, H, n0):
    return pl.kernel(
        functools.partial(_agg_body, fd, H, n0),
        out_type=jax.ShapeDtypeStruct((H, NC, NP, fd), f32),
        mesh=plsc.VectorSubcoreMesh(core_axis_name="c", subcore_axis_name="s"),
        compiler_params=pltpu.CompilerParams(needs_layout_passes=False,
                                             use_tc_tiling_on_sc=False),
        scratch_types=[
            pltpu.VMEM((n0, K), i32),
            pltpu.VMEM((n0, K), i32),
            pltpu.VMEM((n0, K), f32),
            pltpu.VMEM((K, fd), f32),
            pltpu.VMEM((K, fd), f32),
            pltpu.VMEM((K, fd), f32),
            pltpu.VMEM((K, fd), f32),
            pltpu.VMEM((K, fd), f32),
            pltpu.VMEM_SHARED((NP, fd), f32),
            pltpu.SemaphoreType.DMA,
            pltpu.SemaphoreType.DMA,
            pltpu.SemaphoreType.DMA,
            pltpu.SemaphoreType.DMA,
        ],
    )


FH = F // 2
_agg1_call = _make_agg(FH, 2, 80)
_agg2_call = _make_agg(CP, 1, 80)


def _ln(xb, g, b):
    mu = jnp.mean(xb, axis=1, keepdims=True)
    xc = xb - mu
    var = jnp.mean(xc * xc, axis=1, keepdims=True)
    return xc * lax.rsqrt(var + 1e-5) * g + b


def _tcd_body(degp_ref, o_ref):
    o_ref[...] = lax.rsqrt(1.0 + jnp.sum(degp_ref[...], axis=0,
                                         keepdims=True))


def _tc1_body(x_ref, g_ref, b_ref, w_ref, o_ref):
    h = _ln(x_ref[...], g_ref[...], b_ref[...])
    o_ref[...] = lax.dot_general(h, w_ref[...], (((1,), (1,)), ((), ())),
                                 preferred_element_type=f32)


def _tc2_body(dinv_ref, p_ref, o_ref):
    o_ref[...] = p_ref[...] * dinv_ref[...]


def _tc3_body(dinv_ref, agg_ref, q_ref, b1_ref, g_ref, bl_ref, w2_ref, o_ref):
    dinv = dinv_ref[...]
    agg = jnp.concatenate(
        [agg_ref[0, 0] + agg_ref[0, 1], agg_ref[1, 0] + agg_ref[1, 1]], axis=1)
    h = jnp.maximum(dinv * (agg + q_ref[...]) + b1_ref[...], 0.0)
    h = _ln(h, g_ref[...], bl_ref[...])
    p2 = lax.dot_general(h, w2_ref[...], (((1,), (1,)), ((), ())),
                         preferred_element_type=f32)
    o_ref[...] = p2 * dinv


def _tc4_body(dinv_ref, agg_ref, q_ref, b2_ref, o_ref):
    dinv = dinv_ref[...]
    o = dinv * (agg_ref[0, 0] + agg_ref[0, 1] + q_ref[...])
    o = o[:, :C] + b2_ref[...]
    m = jnp.max(o, axis=1, keepdims=True)
    sh = o - m
    o_ref[...] = sh - jnp.log(jnp.sum(jnp.exp(sh), axis=1, keepdims=True))


def _row_spec(fd):
    return pl.BlockSpec((RB, fd), lambda i: (i, 0))


_deg_spec = pl.BlockSpec((RB, 1), lambda i: (i, 0))
_vec_spec_f = pl.BlockSpec((1, F), lambda i: (0, 0))

DCB = 1280

_tcd_call = pl.pallas_call(
    _tcd_body, grid=(NP // DCB,),
    in_specs=[pl.BlockSpec((NW, DCB), lambda j: (0, j))],
    out_specs=pl.BlockSpec((1, DCB), lambda j: (0, j)),
    out_shape=jax.ShapeDtypeStruct((1, NP), f32),
)

_tc1_call = pl.pallas_call(
    _tc1_body, grid=(G,),
    in_specs=[_row_spec(F), _vec_spec_f, _vec_spec_f,
              pl.BlockSpec((F, F), lambda i: (0, 0))],
    out_specs=_row_spec(F),
    out_shape=jax.ShapeDtypeStruct((N, F), f32),
)

_tc2_call = pl.pallas_call(
    _tc2_body, grid=(G,),
    in_specs=[_deg_spec, _row_spec(F)],
    out_specs=_row_spec(F),
    out_shape=jax.ShapeDtypeStruct((N, F), f32),
)

_tc3_call = pl.pallas_call(
    _tc3_body, grid=(G,),
    in_specs=[_deg_spec,
              pl.BlockSpec((2, NC, RB, FH), lambda i: (0, 0, i, 0)),
              _row_spec(F), _vec_spec_f, _vec_spec_f, _vec_spec_f,
              pl.BlockSpec((CP, F), lambda i: (0, 0))],
    out_specs=_row_spec(CP),
    out_shape=jax.ShapeDtypeStruct((N, CP), f32),
)

_tc4_call = pl.pallas_call(
    _tc4_body, grid=(G,),
    in_specs=[_deg_spec,
              pl.BlockSpec((1, NC, RB, CP), lambda i: (0, 0, i, 0)),
              _row_spec(CP),
              pl.BlockSpec((1, C), lambda i: (0, 0))],
    out_specs=_row_spec(C),
    out_shape=jax.ShapeDtypeStruct((N, C), f32),
)


def kernel(x, edge_index, edge_weight, ln0_g, ln0_b, W1, b1,
           ln1_g, ln1_b, W2, b2):
    pad = EPAD - E
    src = jnp.concatenate([edge_index[0], jnp.zeros((pad,), i32)])
    dst = jnp.concatenate([edge_index[1], jnp.zeros((pad,), i32)])
    ew = jnp.concatenate([edge_weight, jnp.zeros((pad,), f32)])
    srcp = src.reshape(TOTCP, K)
    dstp = dst.reshape(TOTCP, K)
    ewp = ew.reshape(TOTCP, K)
    W2p = jnp.concatenate([W2, jnp.zeros((CP - C, F), f32)], axis=0)

    degp = _deg_call(dstp, ewp).reshape(NW, NP)
    dinv = _tcd_call(degp).reshape(NP, 1)
    p1 = _tc1_call(x, ln0_g.reshape(1, F), ln0_b.reshape(1, F), W1)
    q1 = _tc2_call(dinv, p1)
    agg1 = _agg1_call(q1[:, :FH], q1[:, FH:], srcp, dstp, ewp)
    q2 = _tc3_call(dinv, agg1, q1, b1.reshape(1, F), ln1_g.reshape(1, F),
                   ln1_b.reshape(1, F), W2p)
    agg2 = _agg2_call(q2, srcp, dstp, ewp)
    return _tc4_call(dinv, agg2, q2, b2.reshape(1, C))


# speedup vs baseline: 1.5681x; 1.5673x over previous
import functools
import jax
import jax.numpy as jnp
from jax import lax
from jax.experimental import pallas as pl
from jax.experimental.pallas import tpu as pltpu
from jax.experimental.pallas import tpu_sc as plsc

N = 10000
E = 320000
F = 128
C = 40
CP = 48

NC = 2
NS = 16
NW = NC * NS
K = 128
CH = 80
CPP = 2 * CH
TOTC = NW * CH
TOTCP = 2624
EPAD = TOTCP * K
NP = 10240
RPS = NP // NS
CHD = TOTCP // NW

RB = 400
G = N // RB

f32 = jnp.float32
i32 = jnp.int32


def _deg_body(dst_hbm, ew_hbm, out_hbm, dst_v, ew_v, acc_v):
    c = lax.axis_index("c")
    s = lax.axis_index("s")
    wid = s * NC + c
    pltpu.sync_copy(dst_hbm.at[pl.ds(wid * CHD, CHD)], dst_v)
    pltpu.sync_copy(ew_hbm.at[pl.ds(wid * CHD, CHD)], ew_v)

    def init(i, carry):
        acc_v[pl.ds(i * 16, 16)] = jnp.zeros((16,), f32)
        return carry
    lax.fori_loop(0, NP // 16, init, 0)

    def chunk(ci, carry):
        def grp(g, carry2):
            d16 = dst_v[ci, pl.ds(g * 16, 16)]
            w16 = ew_v[ci, pl.ds(g * 16, 16)]
            plsc.addupdate_scatter(acc_v, [d16], w16)
            return carry2
        return lax.fori_loop(0, K // 16, grp, carry)
    lax.fori_loop(0, CHD, chunk, 0)

    pltpu.sync_copy(acc_v, out_hbm.at[wid, 0])


_deg_call = pl.kernel(
    _deg_body,
    out_type=jax.ShapeDtypeStruct((NW, 1, NP), f32),
    mesh=plsc.VectorSubcoreMesh(core_axis_name="c", subcore_axis_name="s"),
    compiler_params=pltpu.CompilerParams(needs_layout_passes=False,
                                         use_tc_tiling_on_sc=False),
    scratch_types=[
        pltpu.VMEM((CHD, K), i32),
        pltpu.VMEM((CHD, K), f32),
        pltpu.VMEM((NP,), f32),
    ],
)


def _agg_body(fd, H, n0, half, *refs):
    qs = refs[:H]
    (src_hbm, dst_hbm, ew_hbm, out_hbm, src_v, dst_v, ew_v,
     rows0, rows1, msg0, msg1, zbuf, acc_sh, gs0, gs1, ss0, ss1) = refs[H:]
    rows = (rows0, rows1)
    msg = (msg0, msg1)
    gsem = (gs0, gs1)
    ssem = (ss0, ss1)
    c = lax.axis_index("c")
    s = lax.axis_index("s")
    fb_n = fd // 16
    n1 = CPP - n0
    if n0 == n1:
        base = (s * NC + c) * n0
        nch = n0
    else:
        base = jnp.where(c == 0, s * n0, NS * n0 + s * n1)
        nch = jnp.where(c == 0, n0, n1)

    pltpu.sync_copy(src_hbm.at[pl.ds(base, n0)], src_v)
    pltpu.sync_copy(dst_hbm.at[pl.ds(base, n0)], dst_v)
    pltpu.sync_copy(ew_hbm.at[pl.ds(base, n0)], ew_v)

    def zrow(r, carry):
        for fb in range(fb_n):
            zbuf[r, pl.ds(fb * 16, 16)] = jnp.zeros((16,), f32)
        return carry
    lax.fori_loop(0, K, zrow, 0)

    for h in range(H):
        for j in range(RPS // K):
            pltpu.sync_copy(zbuf, acc_sh.at[pl.ds(s * RPS + j * K, K)])
        plsc.subcore_barrier()

        for b in range(2):
            pltpu.async_copy(qs[h].at[src_v.at[b]], rows[b], gsem[b])

        def pair(p, carry):
            for b in range(2):
                ci = 2 * p + b
                pltpu.make_async_copy(qs[h].at[src_v.at[ci]],
                                      rows[b], gsem[b]).wait()
                @pl.when(ci >= 2)
                def _drain():
                    pltpu.make_async_copy(
                        msg[b], acc_sh.at[dst_v.at[ci - 2]], ssem[b]).wait()

                iota2 = 2 * lax.iota(i32, 16)

                @plsc.parallel_loop(0, K, unroll=8)
                def _scale(e):
                    splat = plsc.load_gather(
                        ew_v,
                        [jnp.full((16,), ci, i32), jnp.full((16,), e, i32)])
                    if half:
                        row = msg[b].at[e]
                        for g in range(fd // 32):
                            x = rows[b][e, pl.ds(g * 32, 32)]
                            xi = plsc.bitcast(x, i32)
                            ev = plsc.bitcast(
                                lax.shift_left(xi, jnp.int32(16)), f32)
                            od = plsc.bitcast(
                                xi & jnp.int32(-65536), f32)
                            plsc.store_scatter(row, [g * 32 + iota2],
                                               ev * splat)
                            plsc.store_scatter(row, [g * 32 + iota2 + 1],
                                               od * splat)
                    else:
                        for fb in range(fb_n):
                            sl = pl.ds(fb * 16, 16)
                            msg[b][e, sl] = splat * rows[b][e, sl]

                @pl.when(ci + 2 < nch)
                def _prefetch():
                    pltpu.async_copy(qs[h].at[src_v.at[ci + 2]],
                                     rows[b], gsem[b])

                pltpu.async_copy(msg[b], acc_sh.at[dst_v.at[ci]], ssem[b],
                                 add=True)
            return carry
        lax.fori_loop(0, nch // 2, pair, 0)
        for b in range(2):
            pltpu.make_async_copy(msg[b], acc_sh.at[dst_v.at[nch - 2 + b]],
                                  ssem[b]).wait()

        plsc.subcore_barrier()
        pltpu.sync_copy(acc_sh.at[pl.ds(s * RPS, RPS)],
                        out_hbm.at[h, c, pl.ds(s * RPS, RPS)])
        plsc.subcore_barrier()


def _make_agg(fd, H, n0, half):
    rdt = jnp.bfloat16 if half else f32
    return pl.kernel(
        functools.partial(_agg_body, fd, H, n0, half),
        out_type=jax.ShapeDtypeStruct((H, NC, NP, fd), f32),
        mesh=plsc.VectorSubcoreMesh(core_axis_name="c", subcore_axis_name="s"),
        compiler_params=pltpu.CompilerParams(needs_layout_passes=False,
                                             use_tc_tiling_on_sc=False),
        scratch_types=[
            pltpu.VMEM((n0, K), i32),
            pltpu.VMEM((n0, K), i32),
            pltpu.VMEM((n0, K), f32),
            pltpu.VMEM((K, fd), rdt),
            pltpu.VMEM((K, fd), rdt),
            pltpu.VMEM((K, fd), f32),
            pltpu.VMEM((K, fd), f32),
            pltpu.VMEM((K, fd), f32),
            pltpu.VMEM_SHARED((NP, fd), f32),
            pltpu.SemaphoreType.DMA,
            pltpu.SemaphoreType.DMA,
            pltpu.SemaphoreType.DMA,
            pltpu.SemaphoreType.DMA,
        ],
    )


FH = F // 2
_agg1_call = _make_agg(FH, 2, 80, True)
_agg2_call = _make_agg(CP, 1, 80, False)


def _ln(xb, g, b):
    mu = jnp.mean(xb, axis=1, keepdims=True)
    xc = xb - mu
    var = jnp.mean(xc * xc, axis=1, keepdims=True)
    return xc * lax.rsqrt(var + 1e-5) * g + b


def _tcd_body(degp_ref, o_ref):
    o_ref[...] = lax.rsqrt(1.0 + jnp.sum(degp_ref[...], axis=0,
                                         keepdims=True))


def _tc1_body(x_ref, g_ref, b_ref, w_ref, o_ref):
    h = _ln(x_ref[...], g_ref[...], b_ref[...])
    o_ref[...] = lax.dot_general(h, w_ref[...], (((1,), (1,)), ((), ())),
                                 preferred_element_type=f32)


def _tc2_body(dinv_ref, p_ref, o_ref, oh_ref):
    q = p_ref[...] * dinv_ref[...]
    o_ref[...] = q
    oh_ref[...] = q.astype(jnp.bfloat16)


def _tc3_body(dinv_ref, agg_ref, q_ref, b1_ref, g_ref, bl_ref, w2_ref, o_ref):
    dinv = dinv_ref[...]
    agg = jnp.concatenate(
        [agg_ref[0, 0] + agg_ref[0, 1], agg_ref[1, 0] + agg_ref[1, 1]], axis=1)
    h = jnp.maximum(dinv * (agg + q_ref[...]) + b1_ref[...], 0.0)
    h = _ln(h, g_ref[...], bl_ref[...])
    p2 = lax.dot_general(h, w2_ref[...], (((1,), (1,)), ((), ())),
                         preferred_element_type=f32)
    o_ref[...] = p2 * dinv


def _tc4_body(dinv_ref, agg_ref, q_ref, b2_ref, o_ref):
    dinv = dinv_ref[...]
    o = dinv * (agg_ref[0, 0] + agg_ref[0, 1] + q_ref[...])
    o = o[:, :C] + b2_ref[...]
    m = jnp.max(o, axis=1, keepdims=True)
    sh = o - m
    o_ref[...] = sh - jnp.log(jnp.sum(jnp.exp(sh), axis=1, keepdims=True))


def _row_spec(fd):
    return pl.BlockSpec((RB, fd), lambda i: (i, 0))


_deg_spec = pl.BlockSpec((RB, 1), lambda i: (i, 0))
_vec_spec_f = pl.BlockSpec((1, F), lambda i: (0, 0))

DCB = 1280

_tcd_call = pl.pallas_call(
    _tcd_body, grid=(NP // DCB,),
    in_specs=[pl.BlockSpec((NW, DCB), lambda j: (0, j))],
    out_specs=pl.BlockSpec((1, DCB), lambda j: (0, j)),
    out_shape=jax.ShapeDtypeStruct((1, NP), f32),
)

_tc1_call = pl.pallas_call(
    _tc1_body, grid=(G,),
    in_specs=[_row_spec(F), _vec_spec_f, _vec_spec_f,
              pl.BlockSpec((F, F), lambda i: (0, 0))],
    out_specs=_row_spec(F),
    out_shape=jax.ShapeDtypeStruct((N, F), f32),
)

_tc2_call = pl.pallas_call(
    _tc2_body, grid=(G,),
    in_specs=[_deg_spec, _row_spec(F)],
    out_specs=[_row_spec(F), _row_spec(F)],
    out_shape=[jax.ShapeDtypeStruct((N, F), f32),
               jax.ShapeDtypeStruct((N, F), jnp.bfloat16)],
)

_tc3_call = pl.pallas_call(
    _tc3_body, grid=(G,),
    in_specs=[_deg_spec,
              pl.BlockSpec((2, NC, RB, FH), lambda i: (0, 0, i, 0)),
              _row_spec(F), _vec_spec_f, _vec_spec_f, _vec_spec_f,
              pl.BlockSpec((CP, F), lambda i: (0, 0))],
    out_specs=_row_spec(CP),
    out_shape=jax.ShapeDtypeStruct((N, CP), f32),
)

_tc4_call = pl.pallas_call(
    _tc4_body, grid=(G,),
    in_specs=[_deg_spec,
              pl.BlockSpec((1, NC, RB, CP), lambda i: (0, 0, i, 0)),
              _row_spec(CP),
              pl.BlockSpec((1, C), lambda i: (0, 0))],
    out_specs=_row_spec(C),
    out_shape=jax.ShapeDtypeStruct((N, C), f32),
)


def kernel(x, edge_index, edge_weight, ln0_g, ln0_b, W1, b1,
           ln1_g, ln1_b, W2, b2):
    pad = EPAD - E
    src = jnp.concatenate([edge_index[0], jnp.zeros((pad,), i32)])
    dst = jnp.concatenate([edge_index[1], jnp.zeros((pad,), i32)])
    ew = jnp.concatenate([edge_weight, jnp.zeros((pad,), f32)])
    srcp = src.reshape(TOTCP, K)
    dstp = dst.reshape(TOTCP, K)
    ewp = ew.reshape(TOTCP, K)
    W2p = jnp.concatenate([W2, jnp.zeros((CP - C, F), f32)], axis=0)

    degp = _deg_call(dstp, ewp).reshape(NW, NP)
    dinv = _tcd_call(degp).reshape(NP, 1)
    p1 = _tc1_call(x, ln0_g.reshape(1, F), ln0_b.reshape(1, F), W1)
    q1, q1h = _tc2_call(dinv, p1)
    agg1 = _agg1_call(q1h[:, :FH], q1h[:, FH:], srcp, dstp, ewp)
    q2 = _tc3_call(dinv, agg1, q1, b1.reshape(1, F), ln1_g.reshape(1, F),
                   ln1_b.reshape(1, F), W2p)
    agg2 = _agg2_call(q2, srcp, dstp, ewp)
    return _tc4_call(dinv, agg2, q2, b2.reshape(1, C))


# speedup vs baseline: 1.7557x; 1.1196x over previous
import functools
import jax
import jax.numpy as jnp
from jax import lax
from jax.experimental import pallas as pl
from jax.experimental.pallas import tpu as pltpu
from jax.experimental.pallas import tpu_sc as plsc

N = 10000
E = 320000
F = 128
C = 40
CP = 64

NC = 2
NS = 16
NW = NC * NS
K = 128
CH = 80
CPP = 2 * CH
TOTC = NW * CH
TOTCP = 2624
EPAD = TOTCP * K
NP = 10240
RPS = NP // NS
CHD = TOTCP // NW

RB = 400
G = N // RB

f32 = jnp.float32
i32 = jnp.int32


def _deg_body(dst_hbm, ew_hbm, out_hbm, dst_v, ew_v, acc_v):
    c = lax.axis_index("c")
    s = lax.axis_index("s")
    wid = s * NC + c
    pltpu.sync_copy(dst_hbm.at[pl.ds(wid * CHD, CHD)], dst_v)
    pltpu.sync_copy(ew_hbm.at[pl.ds(wid * CHD, CHD)], ew_v)

    def init(i, carry):
        acc_v[pl.ds(i * 16, 16)] = jnp.zeros((16,), f32)
        return carry
    lax.fori_loop(0, NP // 16, init, 0)

    def chunk(ci, carry):
        def grp(g, carry2):
            d16 = dst_v[ci, pl.ds(g * 16, 16)]
            w16 = ew_v[ci, pl.ds(g * 16, 16)]
            plsc.addupdate_scatter(acc_v, [d16], w16)
            return carry2
        return lax.fori_loop(0, K // 16, grp, carry)
    lax.fori_loop(0, CHD, chunk, 0)

    pltpu.sync_copy(acc_v, out_hbm.at[wid, 0])


_deg_call = pl.kernel(
    _deg_body,
    out_type=jax.ShapeDtypeStruct((NW, 1, NP), f32),
    mesh=plsc.VectorSubcoreMesh(core_axis_name="c", subcore_axis_name="s"),
    compiler_params=pltpu.CompilerParams(needs_layout_passes=False,
                                         use_tc_tiling_on_sc=False),
    scratch_types=[
        pltpu.VMEM((CHD, K), i32),
        pltpu.VMEM((CHD, K), f32),
        pltpu.VMEM((NP,), f32),
    ],
)


def _agg_body(fd, H, n0, half, *refs):
    qs = refs[:H]
    (src_hbm, dst_hbm, ew_hbm, out_hbm, src_v, dst_v, ew_v,
     rows0, rows1, msg0, msg1, zbuf, acc_sh, gs0, gs1, ss0, ss1) = refs[H:]
    rows = (rows0, rows1)
    msg = (msg0, msg1)
    gsem = (gs0, gs1)
    ssem = (ss0, ss1)
    c = lax.axis_index("c")
    s = lax.axis_index("s")
    fb_n = fd // 16
    n1 = CPP - n0
    if n0 == n1:
        base = (s * NC + c) * n0
        nch = n0
    else:
        base = jnp.where(c == 0, s * n0, NS * n0 + s * n1)
        nch = jnp.where(c == 0, n0, n1)

    pltpu.sync_copy(src_hbm.at[pl.ds(base, n0)], src_v)
    pltpu.sync_copy(dst_hbm.at[pl.ds(base, n0)], dst_v)
    pltpu.sync_copy(ew_hbm.at[pl.ds(base, n0)], ew_v)

    def zrow(r, carry):
        for fb in range(fb_n):
            zbuf[r, pl.ds(fb * 16, 16)] = jnp.zeros((16,), f32)
        return carry
    lax.fori_loop(0, K, zrow, 0)

    for h in range(H):
        for j in range(RPS // K):
            pltpu.sync_copy(zbuf, acc_sh.at[pl.ds(s * RPS + j * K, K)])
        plsc.subcore_barrier()

        for b in range(2):
            pltpu.async_copy(qs[h].at[src_v.at[b]], rows[b], gsem[b])

        def pair(p, carry):
            for b in range(2):
                ci = 2 * p + b
                pltpu.make_async_copy(qs[h].at[src_v.at[ci]],
                                      rows[b], gsem[b]).wait()
                @pl.when(ci >= 2)
                def _drain():
                    pltpu.make_async_copy(
                        msg[b], acc_sh.at[dst_v.at[ci - 2]], ssem[b]).wait()

                iota2 = 2 * lax.iota(i32, 16)

                @plsc.parallel_loop(0, K, unroll=8)
                def _scale(e):
                    splat = plsc.load_gather(
                        ew_v,
                        [jnp.full((16,), ci, i32), jnp.full((16,), e, i32)])
                    if half:
                        row = msg[b].at[e]
                        for g in range(fd // 32):
                            x = rows[b][e, pl.ds(g * 32, 32)]
                            xi = plsc.bitcast(x, i32)
                            ev = plsc.bitcast(
                                lax.shift_left(xi, jnp.int32(16)), f32)
                            od = plsc.bitcast(
                                xi & jnp.int32(-65536), f32)
                            plsc.store_scatter(row, [g * 32 + iota2],
                                               ev * splat)
                            plsc.store_scatter(row, [g * 32 + iota2 + 1],
                                               od * splat)
                    else:
                        for fb in range(fb_n):
                            sl = pl.ds(fb * 16, 16)
                            msg[b][e, sl] = splat * rows[b][e, sl]

                @pl.when(ci + 2 < nch)
                def _prefetch():
                    pltpu.async_copy(qs[h].at[src_v.at[ci + 2]],
                                     rows[b], gsem[b])

                pltpu.async_copy(msg[b], acc_sh.at[dst_v.at[ci]], ssem[b],
                                 add=True)
            return carry
        lax.fori_loop(0, nch // 2, pair, 0)
        for b in range(2):
            pltpu.make_async_copy(msg[b], acc_sh.at[dst_v.at[nch - 2 + b]],
                                  ssem[b]).wait()

        plsc.subcore_barrier()
        pltpu.sync_copy(acc_sh.at[pl.ds(s * RPS, RPS)],
                        out_hbm.at[h, c, pl.ds(s * RPS, RPS)])
        plsc.subcore_barrier()


def _make_agg(fd, H, n0, half):
    rdt = jnp.bfloat16 if half else f32
    return pl.kernel(
        functools.partial(_agg_body, fd, H, n0, half),
        out_type=jax.ShapeDtypeStruct((H, NC, NP, fd), f32),
        mesh=plsc.VectorSubcoreMesh(core_axis_name="c", subcore_axis_name="s"),
        compiler_params=pltpu.CompilerParams(needs_layout_passes=False,
                                             use_tc_tiling_on_sc=False),
        scratch_types=[
            pltpu.VMEM((n0, K), i32),
            pltpu.VMEM((n0, K), i32),
            pltpu.VMEM((n0, K), f32),
            pltpu.VMEM((K, fd), rdt),
            pltpu.VMEM((K, fd), rdt),
            pltpu.VMEM((K, fd), f32),
            pltpu.VMEM((K, fd), f32),
            pltpu.VMEM((K, fd), f32),
            pltpu.VMEM_SHARED((NP, fd), f32),
            pltpu.SemaphoreType.DMA,
            pltpu.SemaphoreType.DMA,
            pltpu.SemaphoreType.DMA,
            pltpu.SemaphoreType.DMA,
        ],
    )


FH = F // 2
_agg1_call = _make_agg(FH, 2, 80, True)
_agg2_call = _make_agg(CP, 1, 80, True)


def _ln(xb, g, b):
    mu = jnp.mean(xb, axis=1, keepdims=True)
    xc = xb - mu
    var = jnp.mean(xc * xc, axis=1, keepdims=True)
    return xc * lax.rsqrt(var + 1e-5) * g + b


def _tcd_body(degp_ref, o_ref):
    o_ref[...] = lax.rsqrt(1.0 + jnp.sum(degp_ref[...], axis=0,
                                         keepdims=True))


def _tc1_body(x_ref, g_ref, b_ref, w_ref, o_ref):
    h = _ln(x_ref[...], g_ref[...], b_ref[...])
    o_ref[...] = lax.dot_general(h, w_ref[...], (((1,), (1,)), ((), ())),
                                 preferred_element_type=f32)


def _tc2_body(dinv_ref, p_ref, o_ref, oh_ref):
    q = p_ref[...] * dinv_ref[...]
    o_ref[...] = q
    oh_ref[...] = q.astype(jnp.bfloat16)


def _tc3_body(dinv_ref, agg_ref, q_ref, b1_ref, g_ref, bl_ref, w2_ref,
              o_ref, oh_ref):
    dinv = dinv_ref[...]
    agg = jnp.concatenate(
        [agg_ref[0, 0] + agg_ref[0, 1], agg_ref[1, 0] + agg_ref[1, 1]], axis=1)
    h = jnp.maximum(dinv * (agg + q_ref[...]) + b1_ref[...], 0.0)
    h = _ln(h, g_ref[...], bl_ref[...])
    p2 = lax.dot_general(h, w2_ref[...], (((1,), (1,)), ((), ())),
                         preferred_element_type=f32)
    q2 = p2 * dinv
    o_ref[...] = q2
    oh_ref[...] = q2.astype(jnp.bfloat16)


def _tc4_body(dinv_ref, agg_ref, q_ref, b2_ref, o_ref):
    dinv = dinv_ref[...]
    o = dinv * (agg_ref[0, 0] + agg_ref[0, 1] + q_ref[...])
    o = o[:, :C] + b2_ref[...]
    m = jnp.max(o, axis=1, keepdims=True)
    sh = o - m
    o_ref[...] = sh - jnp.log(jnp.sum(jnp.exp(sh), axis=1, keepdims=True))


def _row_spec(fd):
    return pl.BlockSpec((RB, fd), lambda i: (i, 0))


_deg_spec = pl.BlockSpec((RB, 1), lambda i: (i, 0))
_vec_spec_f = pl.BlockSpec((1, F), lambda i: (0, 0))

DCB = 1280

_tcd_call = pl.pallas_call(
    _tcd_body, grid=(NP // DCB,),
    in_specs=[pl.BlockSpec((NW, DCB), lambda j: (0, j))],
    out_specs=pl.BlockSpec((1, DCB), lambda j: (0, j)),
    out_shape=jax.ShapeDtypeStruct((1, NP), f32),
)

_tc1_call = pl.pallas_call(
    _tc1_body, grid=(G,),
    in_specs=[_row_spec(F), _vec_spec_f, _vec_spec_f,
              pl.BlockSpec((F, F), lambda i: (0, 0))],
    out_specs=_row_spec(F),
    out_shape=jax.ShapeDtypeStruct((N, F), f32),
)

_tc2_call = pl.pallas_call(
    _tc2_body, grid=(G,),
    in_specs=[_deg_spec, _row_spec(F)],
    out_specs=[_row_spec(F), _row_spec(F)],
    out_shape=[jax.ShapeDtypeStruct((N, F), f32),
               jax.ShapeDtypeStruct((N, F), jnp.bfloat16)],
)

_tc3_call = pl.pallas_call(
    _tc3_body, grid=(G,),
    in_specs=[_deg_spec,
              pl.BlockSpec((2, NC, RB, FH), lambda i: (0, 0, i, 0)),
              _row_spec(F), _vec_spec_f, _vec_spec_f, _vec_spec_f,
              pl.BlockSpec((CP, F), lambda i: (0, 0))],
    out_specs=[_row_spec(CP), _row_spec(CP)],
    out_shape=[jax.ShapeDtypeStruct((N, CP), f32),
               jax.ShapeDtypeStruct((N, CP), jnp.bfloat16)],
)

_tc4_call = pl.pallas_call(
    _tc4_body, grid=(G,),
    in_specs=[_deg_spec,
              pl.BlockSpec((1, NC, RB, CP), lambda i: (0, 0, i, 0)),
              _row_spec(CP),
              pl.BlockSpec((1, C), lambda i: (0, 0))],
    out_specs=_row_spec(C),
    out_shape=jax.ShapeDtypeStruct((N, C), f32),
)


def kernel(x, edge_index, edge_weight, ln0_g, ln0_b, W1, b1,
           ln1_g, ln1_b, W2, b2):
    pad = EPAD - E
    src = jnp.concatenate([edge_index[0], jnp.zeros((pad,), i32)])
    dst = jnp.concatenate([edge_index[1], jnp.zeros((pad,), i32)])
    ew = jnp.concatenate([edge_weight, jnp.zeros((pad,), f32)])
    srcp = src.reshape(TOTCP, K)
    dstp = dst.reshape(TOTCP, K)
    ewp = ew.reshape(TOTCP, K)
    W2p = jnp.concatenate([W2, jnp.zeros((CP - C, F), f32)], axis=0)

    degp = _deg_call(dstp, ewp).reshape(NW, NP)
    dinv = _tcd_call(degp).reshape(NP, 1)
    p1 = _tc1_call(x, ln0_g.reshape(1, F), ln0_b.reshape(1, F), W1)
    q1, q1h = _tc2_call(dinv, p1)
    agg1 = _agg1_call(q1h[:, :FH], q1h[:, FH:], srcp, dstp, ewp)
    q2, q2h = _tc3_call(dinv, agg1, q1, b1.reshape(1, F), ln1_g.reshape(1, F),
                        ln1_b.reshape(1, F), W2p)
    agg2 = _agg2_call(q2h, srcp, dstp, ewp)
    return _tc4_call(dinv, agg2, q2, b2.reshape(1, C))


# speedup vs baseline: 1.8190x; 1.0360x over previous
import functools
import jax
import jax.numpy as jnp
from jax import lax
from jax.experimental import pallas as pl
from jax.experimental.pallas import tpu as pltpu
from jax.experimental.pallas import tpu_sc as plsc

N = 10000
E = 320000
F = 128
C = 40
CP = 64

NC = 2
NS = 16
NW = NC * NS
K = 128
CH = 80
CPP = 2 * CH
TOTC = NW * CH
TOTCP = 2624
EPAD = TOTCP * K
NP = 10240
RPS = NP // NS
CHD = TOTCP // NW

RB = 400
G = N // RB

f32 = jnp.float32
i32 = jnp.int32


def _deg_body(dst_hbm, ew_hbm, out_hbm, dst_v, ew_v, acc_v):
    c = lax.axis_index("c")
    s = lax.axis_index("s")
    wid = s * NC + c
    pltpu.sync_copy(dst_hbm.at[pl.ds(wid * CHD, CHD)], dst_v)
    pltpu.sync_copy(ew_hbm.at[pl.ds(wid * CHD, CHD)], ew_v)

    def init(i, carry):
        acc_v[pl.ds(i * 16, 16)] = jnp.zeros((16,), f32)
        return carry
    lax.fori_loop(0, NP // 16, init, 0)

    def chunk(ci, carry):
        def grp(g, carry2):
            d16 = dst_v[ci, pl.ds(g * 16, 16)]
            w16 = ew_v[ci, pl.ds(g * 16, 16)]
            plsc.addupdate_scatter(acc_v, [d16], w16)
            return carry2
        return lax.fori_loop(0, K // 16, grp, carry)
    lax.fori_loop(0, CHD, chunk, 0)

    pltpu.sync_copy(acc_v, out_hbm.at[wid, 0])


_deg_call = pl.kernel(
    _deg_body,
    out_type=jax.ShapeDtypeStruct((NW, 1, NP), f32),
    mesh=plsc.VectorSubcoreMesh(core_axis_name="c", subcore_axis_name="s"),
    compiler_params=pltpu.CompilerParams(needs_layout_passes=False,
                                         use_tc_tiling_on_sc=False),
    scratch_types=[
        pltpu.VMEM((CHD, K), i32),
        pltpu.VMEM((CHD, K), f32),
        pltpu.VMEM((NP,), f32),
    ],
)


def _agg_body(fd, H, n0, half, *refs):
    qs = refs[:H]
    (src_hbm, dst_hbm, ew_hbm, out_hbm, src_v, dst_v, ew_v,
     rows0, rows1, msg0, msg1, zbuf, acc_sh, gs0, gs1, ss0, ss1) = refs[H:]
    rows = (rows0, rows1)
    msg = (msg0, msg1)
    gsem = (gs0, gs1)
    ssem = (ss0, ss1)
    c = lax.axis_index("c")
    s = lax.axis_index("s")
    fb_n = fd // 16
    n1 = CPP - n0
    if n0 == n1:
        base = (s * NC + c) * n0
        nch = n0
    else:
        base = jnp.where(c == 0, s * n0, NS * n0 + s * n1)
        nch = jnp.where(c == 0, n0, n1)

    pltpu.sync_copy(src_hbm.at[pl.ds(base, n0)], src_v)
    pltpu.sync_copy(dst_hbm.at[pl.ds(base, n0)], dst_v)
    pltpu.sync_copy(ew_hbm.at[pl.ds(base, n0)], ew_v)

    def zrow(r, carry):
        for fb in range(fb_n):
            zbuf[r, pl.ds(fb * 16, 16)] = jnp.zeros((16,), f32)
        return carry
    lax.fori_loop(0, K, zrow, 0)

    for h in range(H):
        for j in range(RPS // K):
            pltpu.sync_copy(zbuf, acc_sh.at[pl.ds(s * RPS + j * K, K)])
        plsc.subcore_barrier()

        for b in range(2):
            pltpu.async_copy(qs[h].at[src_v.at[b]], rows[b], gsem[b])

        def pair(p, carry):
            for b in range(2):
                ci = 2 * p + b
                pltpu.make_async_copy(qs[h].at[src_v.at[ci]],
                                      rows[b], gsem[b]).wait()
                @pl.when(ci >= 2)
                def _drain():
                    pltpu.make_async_copy(
                        msg[b], acc_sh.at[dst_v.at[ci - 2]], ssem[b]).wait()

                iota2 = 2 * lax.iota(i32, 16)

                @plsc.parallel_loop(0, K, unroll=8)
                def _scale(e):
                    splat = plsc.load_gather(
                        ew_v,
                        [jnp.full((16,), ci, i32), jnp.full((16,), e, i32)])
                    if half:
                        row = msg[b].at[e]
                        for g in range(fd // 32):
                            x = rows[b][e, pl.ds(g * 32, 32)]
                            xi = plsc.bitcast(x, i32)
                            ev = plsc.bitcast(
                                lax.shift_left(xi, jnp.int32(16)), f32)
                            od = plsc.bitcast(
                                xi & jnp.int32(-65536), f32)
                            plsc.store_scatter(row, [g * 32 + iota2],
                                               ev * splat)
                            plsc.store_scatter(row, [g * 32 + iota2 + 1],
                                               od * splat)
                    else:
                        for fb in range(fb_n):
                            sl = pl.ds(fb * 16, 16)
                            msg[b][e, sl] = splat * rows[b][e, sl]

                @pl.when(ci + 2 < nch)
                def _prefetch():
                    pltpu.async_copy(qs[h].at[src_v.at[ci + 2]],
                                     rows[b], gsem[b])

                pltpu.async_copy(msg[b], acc_sh.at[dst_v.at[ci]], ssem[b],
                                 add=True)
            return carry
        lax.fori_loop(0, nch // 2, pair, 0)
        for b in range(2):
            pltpu.make_async_copy(msg[b], acc_sh.at[dst_v.at[nch - 2 + b]],
                                  ssem[b]).wait()

        plsc.subcore_barrier()
        pltpu.sync_copy(acc_sh.at[pl.ds(s * RPS, RPS)],
                        out_hbm.at[h, c, pl.ds(s * RPS, RPS)])
        plsc.subcore_barrier()


def _make_agg(fd, H, n0, half):
    rdt = jnp.bfloat16 if half else f32
    return pl.kernel(
        functools.partial(_agg_body, fd, H, n0, half),
        out_type=jax.ShapeDtypeStruct((H, NC, NP, fd), f32),
        mesh=plsc.VectorSubcoreMesh(core_axis_name="c", subcore_axis_name="s"),
        compiler_params=pltpu.CompilerParams(needs_layout_passes=False,
                                             use_tc_tiling_on_sc=False),
        scratch_types=[
            pltpu.VMEM((n0, K), i32),
            pltpu.VMEM((n0, K), i32),
            pltpu.VMEM((n0, K), f32),
            pltpu.VMEM((K, fd), rdt),
            pltpu.VMEM((K, fd), rdt),
            pltpu.VMEM((K, fd), f32),
            pltpu.VMEM((K, fd), f32),
            pltpu.VMEM((K, fd), f32),
            pltpu.VMEM_SHARED((NP, fd), f32),
            pltpu.SemaphoreType.DMA,
            pltpu.SemaphoreType.DMA,
            pltpu.SemaphoreType.DMA,
            pltpu.SemaphoreType.DMA,
        ],
    )


FH = F // 2
_agg1_call = _make_agg(FH, 2, 96, True)
_agg2_call = _make_agg(CP, 1, 96, True)


def _ln(xb, g, b):
    mu = jnp.mean(xb, axis=1, keepdims=True)
    xc = xb - mu
    var = jnp.mean(xc * xc, axis=1, keepdims=True)
    return xc * lax.rsqrt(var + 1e-5) * g + b


def _tcd_body(degp_ref, o_ref):
    o_ref[...] = lax.rsqrt(1.0 + jnp.sum(degp_ref[...], axis=0,
                                         keepdims=True))


def _tc1_body(x_ref, g_ref, b_ref, w_ref, o_ref):
    h = _ln(x_ref[...], g_ref[...], b_ref[...])
    o_ref[...] = lax.dot_general(h, w_ref[...], (((1,), (1,)), ((), ())),
                                 preferred_element_type=f32)


def _tc2_body(dinv_ref, p_ref, o_ref, oa_ref, ob_ref):
    q = p_ref[...] * dinv_ref[...]
    o_ref[...] = q
    qh = q.astype(jnp.bfloat16)
    oa_ref[...] = qh[:, :FH]
    ob_ref[...] = qh[:, FH:]


def _tc3_body(dinv_ref, agg_ref, q_ref, b1_ref, g_ref, bl_ref, w2_ref,
              o_ref, oh_ref):
    dinv = dinv_ref[...]
    agg = jnp.concatenate(
        [agg_ref[0, 0] + agg_ref[0, 1], agg_ref[1, 0] + agg_ref[1, 1]], axis=1)
    h = jnp.maximum(dinv * (agg + q_ref[...]) + b1_ref[...], 0.0)
    h = _ln(h, g_ref[...], bl_ref[...])
    p2 = lax.dot_general(h, w2_ref[...], (((1,), (1,)), ((), ())),
                         preferred_element_type=f32)
    q2 = p2 * dinv
    o_ref[...] = q2
    oh_ref[...] = q2.astype(jnp.bfloat16)


def _tc4_body(dinv_ref, agg_ref, q_ref, b2_ref, o_ref):
    dinv = dinv_ref[...]
    o = dinv * (agg_ref[0, 0] + agg_ref[0, 1] + q_ref[...])
    o = o[:, :C] + b2_ref[...]
    m = jnp.max(o, axis=1, keepdims=True)
    sh = o - m
    o_ref[...] = sh - jnp.log(jnp.sum(jnp.exp(sh), axis=1, keepdims=True))


def _row_spec(fd):
    return pl.BlockSpec((RB, fd), lambda i: (i, 0))


_deg_spec = pl.BlockSpec((RB, 1), lambda i: (i, 0))
_vec_spec_f = pl.BlockSpec((1, F), lambda i: (0, 0))

DCB = 1280

_tcd_call = pl.pallas_call(
    _tcd_body, grid=(NP // DCB,),
    in_specs=[pl.BlockSpec((NW, DCB), lambda j: (0, j))],
    out_specs=pl.BlockSpec((1, DCB), lambda j: (0, j)),
    out_shape=jax.ShapeDtypeStruct((1, NP), f32),
)

_tc1_call = pl.pallas_call(
    _tc1_body, grid=(G,),
    in_specs=[_row_spec(F), _vec_spec_f, _vec_spec_f,
              pl.BlockSpec((F, F), lambda i: (0, 0))],
    out_specs=_row_spec(F),
    out_shape=jax.ShapeDtypeStruct((N, F), f32),
)

_tc2_call = pl.pallas_call(
    _tc2_body, grid=(G,),
    in_specs=[_deg_spec, _row_spec(F)],
    out_specs=[_row_spec(F), _row_spec(FH), _row_spec(FH)],
    out_shape=[jax.ShapeDtypeStruct((N, F), f32),
               jax.ShapeDtypeStruct((N, FH), jnp.bfloat16),
               jax.ShapeDtypeStruct((N, FH), jnp.bfloat16)],
)

_tc3_call = pl.pallas_call(
    _tc3_body, grid=(G,),
    in_specs=[_deg_spec,
              pl.BlockSpec((2, NC, RB, FH), lambda i: (0, 0, i, 0)),
              _row_spec(F), _vec_spec_f, _vec_spec_f, _vec_spec_f,
              pl.BlockSpec((CP, F), lambda i: (0, 0))],
    out_specs=[_row_spec(CP), _row_spec(CP)],
    out_shape=[jax.ShapeDtypeStruct((N, CP), f32),
               jax.ShapeDtypeStruct((N, CP), jnp.bfloat16)],
)

_tc4_call = pl.pallas_call(
    _tc4_body, grid=(G,),
    in_specs=[_deg_spec,
              pl.BlockSpec((1, NC, RB, CP), lambda i: (0, 0, i, 0)),
              _row_spec(CP),
              pl.BlockSpec((1, C), lambda i: (0, 0))],
    out_specs=_row_spec(C),
    out_shape=jax.ShapeDtypeStruct((N, C), f32),
)


def kernel(x, edge_index, edge_weight, ln0_g, ln0_b, W1, b1,
           ln1_g, ln1_b, W2, b2):
    pad = EPAD - E
    src = jnp.concatenate([edge_index[0], jnp.zeros((pad,), i32)])
    dst = jnp.concatenate([edge_index[1], jnp.zeros((pad,), i32)])
    ew = jnp.concatenate([edge_weight, jnp.zeros((pad,), f32)])
    srcp = src.reshape(TOTCP, K)
    dstp = dst.reshape(TOTCP, K)
    ewp = ew.reshape(TOTCP, K)
    W2p = jnp.concatenate([W2, jnp.zeros((CP - C, F), f32)], axis=0)

    degp = _deg_call(dstp, ewp).reshape(NW, NP)
    dinv = _tcd_call(degp).reshape(NP, 1)
    p1 = _tc1_call(x, ln0_g.reshape(1, F), ln0_b.reshape(1, F), W1)
    q1, qa, qb = _tc2_call(dinv, p1)
    agg1 = _agg1_call(qa, qb, srcp, dstp, ewp)
    q2, q2h = _tc3_call(dinv, agg1, q1, b1.reshape(1, F), ln1_g.reshape(1, F),
                        ln1_b.reshape(1, F), W2p)
    agg2 = _agg2_call(q2h, srcp, dstp, ewp)
    return _tc4_call(dinv, agg2, q2, b2.reshape(1, C))
